# baseline (speedup 1.0000x reference)
import functools

import jax
import jax.numpy as jnp
from jax import lax
from jax.experimental import pallas as pl
from jax.experimental.pallas import tpu as pltpu
from jax.experimental.pallas import tpu_sc as plsc

EMB = 128
NUM_ITEMS = 16384
MAX_SIZE = 20

NC = 2
NS = 16
NW = NC * NS
C = 32
HC = C // 16
ROWS = C * MAX_SIZE
CPW = NUM_ITEMS // (NW * C)
IPW = NUM_ITEMS // NW
NSPLIT = 5
GLEN = ROWS // NSPLIT
NVREG = EMB // 16
D = 4
RW = 2


def _vlane_gather(x, idx):
    dnums = lax.GatherDimensionNumbers(
        offset_dims=(), collapsed_slice_dims=(0,), start_index_map=(0,))
    return lax.gather(x, idx[:, None], dnums, slice_sizes=(1,),
                      mode=lax.GatherScatterMode.PROMISE_IN_BOUNDS)


def _sc_body(idxw_hbm, len_hbm, table_hbm, out_hbm,
             len_all, row0_v, *rest):
    perm = rest[0:D]
    tm = rest[D:2 * D]
    acc = rest[2 * D:3 * D]
    outb = rest[3 * D:4 * D]
    sem_i = rest[4 * D:5 * D]
    sem_r = rest[5 * D:6 * D]
    sem_o = rest[6 * D:7 * D]
    wid = lax.axis_index("s") * NC + lax.axis_index("c")
    chunk0 = wid * CPW

    pltpu.sync_copy(table_hbm.at[pl.ds(0, 1)], row0_v)
    pltpu.sync_copy(len_hbm.at[pl.ds(wid * IPW, IPW)], len_all)

    iota20 = lax.iota(jnp.int32, 16) * MAX_SIZE
    zeros16 = jnp.zeros((16,), jnp.float32)
    row0 = [row0_v[0, pl.ds(v * 16, 16)] for v in range(NVREG)]

    def build_perm(ci, k):
        base = (chunk0 + ci) * ROWS + iota20
        for j in range(MAX_SIZE):
            for h in range(HC):
                perm[k][pl.ds(j * C + h * 16, 16)] = base + (h * 16 * MAX_SIZE + j)

    def issue_idx(k):
        for s in range(NSPLIT):
            pltpu.async_copy(
                idxw_hbm.at[perm[k].at[pl.ds(s * GLEN, GLEN)]],
                tm[k].at[pl.ds(s * GLEN, GLEN)], sem_i[k])

    def drain_idx(k):
        for s in range(NSPLIT):
            pltpu.make_async_copy(
                idxw_hbm.at[perm[k].at[pl.ds(s * GLEN, GLEN)]],
                tm[k].at[pl.ds(s * GLEN, GLEN)], sem_i[k]).wait()

    def issue_rows(k):
        for j in range(MAX_SIZE):
            pltpu.async_copy(
                table_hbm.at[tm[k].at[pl.ds(j * C, C)]],
                acc[k], sem_r[k], add=True)

    def drain_rows(k):
        for j in range(MAX_SIZE):
            pltpu.make_async_copy(
                table_hbm.at[tm[k].at[pl.ds(j * C, C)]],
                acc[k], sem_r[k]).wait()

    def zero_acc(k):
        for i in range(C):
            for v in range(NVREG):
                acc[k][i, pl.ds(v * 16, 16)] = zeros16

    def prep(ci, k):
        out = []
        for h in range(HC):
            zc = jnp.zeros((16,), jnp.float32)
            for j in range(MAX_SIZE):
                tok = tm[k][pl.ds(j * C + h * 16, 16)]
                zc = zc + jnp.where(tok == 0, jnp.float32(1.0),
                                    jnp.float32(0.0))
            rcpv = jnp.float32(1.0) / len_all[
                pl.ds(ci * C + h * 16, 16)].astype(jnp.float32)
            out.append((rcpv, zc * rcpv))
        return out

    def scale(k, halves):
        for h in range(HC):
            rcpv, zrv = halves[h]

            def item_body(i, c2, _h=h, _rcpv=rcpv, _zrv=zrv):
                bidx = jnp.full((16,), i, jnp.int32)
                a = _vlane_gather(_rcpv, bidx)
                b = _vlane_gather(_zrv, bidx)
                for v in range(NVREG):
                    sl = pl.ds(v * 16, 16)
                    row = _h * 16 + i
                    outb[k][row, sl] = acc[k][row, sl] * a - b * row0[v]
                return c2
            lax.fori_loop(0, 16, item_body, 0, unroll=False)

    def issue_out(ci, k):
        pltpu.async_copy(
            outb[k], out_hbm.at[pl.ds((chunk0 + ci) * C, C)], sem_o[k])

    def drain_out(k):
        pltpu.make_async_copy(
            outb[k], out_hbm.at[pl.ds(0, C)], sem_o[k]).wait()

    for k in range(D):
        zero_acc(k)
    for q in range(RW):
        build_perm(q, q)
        issue_idx(q)
    for q in range(RW):
        drain_idx(q)
        issue_rows(q)
        if RW + q < D:
            build_perm(RW + q, RW + q)
            issue_idx(RW + q)

    def group_body(p, carry):
        for k in range(D):
            c = D * p + k
            drain_rows(k)
            halves = prep(c, k)

            @pl.when(p > 0)
            def _():
                drain_out(k)

            scale(k, halves)
            issue_out(c, k)
            zero_acc(k)

            @pl.when(c + RW < CPW)
            def _():
                drain_idx((k + RW) % D)
                issue_rows((k + RW) % D)

            @pl.when(c + D < CPW)
            def _():
                build_perm(c + D, k)
                issue_idx(k)
        return carry

    lax.fori_loop(0, CPW // D, group_body, 0, unroll=False)
    for k in range(D):
        drain_out(k)


def kernel(input_tensor, item_size, emb_table):
    idx_w = input_tensor.reshape(NUM_ITEMS * MAX_SIZE).astype(jnp.int32)
    lens = item_size.astype(jnp.int32)

    mesh = plsc.VectorSubcoreMesh(core_axis_name="c", subcore_axis_name="s")
    run = functools.partial(
        pl.kernel,
        mesh=mesh,
        out_type=jax.ShapeDtypeStruct((NUM_ITEMS, EMB), jnp.float32),
        scratch_types=(
            [pltpu.VMEM((IPW,), jnp.int32),
             pltpu.VMEM((1, EMB), jnp.float32)]
            + [pltpu.VMEM((ROWS,), jnp.int32) for _ in range(D)]
            + [pltpu.VMEM((ROWS,), jnp.int32) for _ in range(D)]
            + [pltpu.VMEM((C, EMB), jnp.float32) for _ in range(D)]
            + [pltpu.VMEM((C, EMB), jnp.float32) for _ in range(D)]
            + [pltpu.SemaphoreType.DMA for _ in range(3 * D)]
        ),
    )(_sc_body)
    return run(idx_w, lens, emb_table)


# speedup vs baseline: 1.0370x; 1.0370x over previous
import functools

import jax
import jax.numpy as jnp
from jax import lax
from jax.experimental import pallas as pl
from jax.experimental.pallas import tpu as pltpu
from jax.experimental.pallas import tpu_sc as plsc

EMB = 128
NUM_ITEMS = 16384
MAX_SIZE = 20

NC = 2
NS = 16
NW = NC * NS
C = 16
ROWS = C * MAX_SIZE
CPW = NUM_ITEMS // (NW * C)
IPW = NUM_ITEMS // NW
NSPLIT = 4
GLEN = ROWS // NSPLIT
NVREG = EMB // 16
D = 8
RW = 6


def _vlane_gather(x, idx):
    dnums = lax.GatherDimensionNumbers(
        offset_dims=(), collapsed_slice_dims=(0,), start_index_map=(0,))
    return lax.gather(x, idx[:, None], dnums, slice_sizes=(1,),
                      mode=lax.GatherScatterMode.PROMISE_IN_BOUNDS)


def _sc_body(idxw_hbm, len_hbm, table_hbm, out_hbm,
             len_all, row0_v, *rest):
    perm = rest[0:D]
    tm = rest[D:2 * D]
    acc = rest[2 * D:3 * D]
    outb = rest[3 * D:4 * D]
    sem_i = rest[4 * D:5 * D]
    sem_r = rest[5 * D:6 * D]
    sem_o = rest[6 * D:7 * D]
    wid = lax.axis_index("s") * NC + lax.axis_index("c")
    chunk0 = wid * CPW

    pltpu.sync_copy(table_hbm.at[pl.ds(0, 1)], row0_v)
    pltpu.sync_copy(len_hbm.at[pl.ds(wid * IPW, IPW)], len_all)

    iota20 = lax.iota(jnp.int32, 16) * MAX_SIZE
    zeros16 = jnp.zeros((16,), jnp.float32)
    row0 = [row0_v[0, pl.ds(v * 16, 16)] for v in range(NVREG)]

    def build_perm(ci, k):
        base = (chunk0 + ci) * ROWS + iota20
        for j in range(MAX_SIZE):
            perm[k][pl.ds(j * 16, 16)] = base + j

    def issue_idx(k):
        for s in range(NSPLIT):
            pltpu.async_copy(
                idxw_hbm.at[perm[k].at[pl.ds(s * GLEN, GLEN)]],
                tm[k].at[pl.ds(s * GLEN, GLEN)], sem_i[k])

    def drain_idx(k):
        for s in range(NSPLIT):
            pltpu.make_async_copy(
                idxw_hbm.at[perm[k].at[pl.ds(s * GLEN, GLEN)]],
                tm[k].at[pl.ds(s * GLEN, GLEN)], sem_i[k]).wait()

    def issue_rows(k):
        for j in range(MAX_SIZE):
            pltpu.async_copy(
                table_hbm.at[tm[k].at[pl.ds(j * 16, 16)]],
                acc[k], sem_r[k], add=True)

    def drain_rows(k):
        for j in range(MAX_SIZE):
            pltpu.make_async_copy(
                table_hbm.at[tm[k].at[pl.ds(j * 16, 16)]],
                acc[k], sem_r[k]).wait()

    def zero_acc(k):
        for i in range(C):
            for v in range(NVREG):
                acc[k][i, pl.ds(v * 16, 16)] = zeros16

    def prep(ci, k):
        zc = jnp.zeros((16,), jnp.float32)
        for j in range(MAX_SIZE):
            tok = tm[k][pl.ds(j * 16, 16)]
            zc = zc + jnp.where(tok == 0, jnp.float32(1.0), jnp.float32(0.0))
        rcpv = jnp.float32(1.0) / len_all[pl.ds(ci * C, C)].astype(jnp.float32)
        return rcpv, zc * rcpv

    def scale(k, rcpv, zrv):
        def item_body(i, c2):
            bidx = jnp.full((16,), i, jnp.int32)
            a = _vlane_gather(rcpv, bidx)
            b = _vlane_gather(zrv, bidx)
            for v in range(NVREG):
                sl = pl.ds(v * 16, 16)
                outb[k][i, sl] = acc[k][i, sl] * a - b * row0[v]
            return c2
        lax.fori_loop(0, C, item_body, 0, unroll=False)

    def issue_out(ci, k):
        pltpu.async_copy(
            outb[k], out_hbm.at[pl.ds((chunk0 + ci) * C, C)], sem_o[k])

    def drain_out(k):
        pltpu.make_async_copy(
            outb[k], out_hbm.at[pl.ds(0, C)], sem_o[k]).wait()

    for k in range(D):
        zero_acc(k)
    for q in range(RW):
        build_perm(q, q)
        issue_idx(q)
    for q in range(RW):
        drain_idx(q)
        issue_rows(q)
        if RW + q < D:
            build_perm(RW + q, RW + q)
            issue_idx(RW + q)

    def group_body(p, carry):
        for k in range(D):
            c = D * p + k
            drain_rows(k)
            rcpv, zrv = prep(c, k)

            @pl.when(p > 0)
            def _():
                drain_out(k)

            scale(k, rcpv, zrv)
            issue_out(c, k)
            zero_acc(k)

            @pl.when(c + RW < CPW)
            def _():
                drain_idx((k + RW) % D)
                issue_rows((k + RW) % D)

            @pl.when(c + D < CPW)
            def _():
                build_perm(c + D, k)
                issue_idx(k)
        return carry

    lax.fori_loop(0, CPW // D, group_body, 0, unroll=False)
    for k in range(D):
        drain_out(k)


def kernel(input_tensor, item_size, emb_table):
    idx_w = input_tensor.reshape(NUM_ITEMS * MAX_SIZE).astype(jnp.int32)
    lens = item_size.astype(jnp.int32)

    mesh = plsc.VectorSubcoreMesh(core_axis_name="c", subcore_axis_name="s")
    run = functools.partial(
        pl.kernel,
        mesh=mesh,
        out_type=jax.ShapeDtypeStruct((NUM_ITEMS, EMB), jnp.float32),
        scratch_types=(
            [pltpu.VMEM((IPW,), jnp.int32),
             pltpu.VMEM((1, EMB), jnp.float32)]
            + [pltpu.VMEM((ROWS,), jnp.int32) for _ in range(D)]
            + [pltpu.VMEM((ROWS,), jnp.int32) for _ in range(D)]
            + [pltpu.VMEM((C, EMB), jnp.float32) for _ in range(D)]
            + [pltpu.VMEM((C, EMB), jnp.float32) for _ in range(D)]
            + [pltpu.SemaphoreType.DMA for _ in range(3 * D)]
        ),
    )(_sc_body)
    return run(idx_w, lens, emb_table)


# speedup vs baseline: 1.3864x; 1.3370x over previous
import functools

import jax
import jax.numpy as jnp
from jax import lax
from jax.experimental import pallas as pl
from jax.experimental.pallas import tpu as pltpu
from jax.experimental.pallas import tpu_sc as plsc

EMB = 128
NUM_ITEMS = 16384
MAX_SIZE = 20

NC = 2
NS = 16
NW = NC * NS
C = 16
CPW = NUM_ITEMS // (NW * C)
IPW = NUM_ITEMS // NW
NVREG = EMB // 16
D = 8
RW = 4


def _vlane_gather(x, idx):
    dnums = lax.GatherDimensionNumbers(
        offset_dims=(), collapsed_slice_dims=(0,), start_index_map=(0,))
    return lax.gather(x, idx[:, None], dnums, slice_sizes=(1,),
                      mode=lax.GatherScatterMode.PROMISE_IN_BOUNDS)


def _sc_body(idxt_hbm, len_hbm, table_hbm, out_hbm,
             len_all, row0_v, tmall, *rest):
    acc = rest[0:D]
    outb = rest[D:2 * D]
    sem_t = rest[2 * D]
    sem_r = rest[2 * D + 1:3 * D + 1]
    sem_o = rest[3 * D + 1:4 * D + 1]
    wid = lax.axis_index("s") * NC + lax.axis_index("c")
    chunk0 = wid * CPW
    item0 = wid * IPW

    pltpu.sync_copy(table_hbm.at[pl.ds(0, 1)], row0_v)
    pltpu.sync_copy(len_hbm.at[pl.ds(item0, IPW)], len_all)
    for j in range(MAX_SIZE):
        pltpu.async_copy(
            idxt_hbm.at[pl.ds(j * NUM_ITEMS + item0, IPW)],
            tmall.at[pl.ds(j * IPW, IPW)], sem_t)

    zeros16 = jnp.zeros((16,), jnp.float32)
    row0 = [row0_v[0, pl.ds(v * 16, 16)] for v in range(NVREG)]

    def tok_list(ci, j):
        return tmall.at[pl.ds(j * IPW, IPW)].at[pl.ds(ci * C, C)]

    def issue_rows(ci, k):
        for j in range(MAX_SIZE):
            pltpu.async_copy(
                table_hbm.at[tok_list(ci, j)],
                acc[k], sem_r[k], add=True)

    def drain_rows(ci, k):
        for j in range(MAX_SIZE):
            pltpu.make_async_copy(
                table_hbm.at[tok_list(ci, j)],
                acc[k], sem_r[k]).wait()

    def zero_acc(k):
        for i in range(C):
            for v in range(NVREG):
                acc[k][i, pl.ds(v * 16, 16)] = zeros16

    def prep(ci):
        zc = jnp.zeros((16,), jnp.float32)
        for j in range(MAX_SIZE):
            tok = tmall[pl.ds(j * IPW + ci * C, C)]
            zc = zc + jnp.where(tok == 0, jnp.float32(1.0), jnp.float32(0.0))
        rcpv = jnp.float32(1.0) / len_all[pl.ds(ci * C, C)].astype(jnp.float32)
        return rcpv, zc * rcpv

    def scale(k, rcpv, zrv):
        def item_body(i, c2):
            bidx = jnp.full((16,), i, jnp.int32)
            a = _vlane_gather(rcpv, bidx)
            b = _vlane_gather(zrv, bidx)
            for v in range(NVREG):
                sl = pl.ds(v * 16, 16)
                outb[k][i, sl] = acc[k][i, sl] * a - b * row0[v]
            return c2
        lax.fori_loop(0, C, item_body, 0, unroll=False)

    def issue_out(ci, k):
        pltpu.async_copy(
            outb[k], out_hbm.at[pl.ds((chunk0 + ci) * C, C)], sem_o[k])

    def drain_out(k):
        pltpu.make_async_copy(
            outb[k], out_hbm.at[pl.ds(0, C)], sem_o[k]).wait()

    for k in range(D):
        zero_acc(k)
    for j in range(MAX_SIZE):
        pltpu.make_async_copy(
            idxt_hbm.at[pl.ds(j * NUM_ITEMS + item0, IPW)],
            tmall.at[pl.ds(j * IPW, IPW)], sem_t).wait()
    for q in range(RW):
        issue_rows(q, q)

    def group_body(p, carry):
        for k in range(D):
            c = D * p + k
            drain_rows(c, k)
            rcpv, zrv = prep(c)

            @pl.when(p > 0)
            def _():
                drain_out(k)

            scale(k, rcpv, zrv)
            issue_out(c, k)
            zero_acc(k)

            @pl.when(c + RW < CPW)
            def _():
                issue_rows(c + RW, (k + RW) % D)
        return carry

    lax.fori_loop(0, CPW // D, group_body, 0, unroll=False)
    for k in range(D):
        drain_out(k)


def kernel(input_tensor, item_size, emb_table):
    idx_t = input_tensor.astype(jnp.int32).T.reshape(MAX_SIZE * NUM_ITEMS)
    lens = item_size.astype(jnp.int32)

    mesh = plsc.VectorSubcoreMesh(core_axis_name="c", subcore_axis_name="s")
    run = functools.partial(
        pl.kernel,
        mesh=mesh,
        out_type=jax.ShapeDtypeStruct((NUM_ITEMS, EMB), jnp.float32),
        scratch_types=(
            [pltpu.VMEM((IPW,), jnp.int32),
             pltpu.VMEM((1, EMB), jnp.float32),
             pltpu.VMEM((MAX_SIZE * IPW,), jnp.int32)]
            + [pltpu.VMEM((C, EMB), jnp.float32) for _ in range(D)]
            + [pltpu.VMEM((C, EMB), jnp.float32) for _ in range(D)]
            + [pltpu.SemaphoreType.DMA]
            + [pltpu.SemaphoreType.DMA for _ in range(2 * D)]
        ),
    )(_sc_body)
    return run(idx_t, lens, emb_table)
